# baseline (speedup 1.0000x reference)
import functools

import jax
import jax.numpy as jnp
from jax import lax
from jax.experimental import pallas as pl
from jax.experimental.pallas import tpu as pltpu
from jax.experimental.pallas import tpu_sc as plsc

_BB = 8
_TCH = 16
_NW = 32


def _attention(x2, w1, w2, b1, b2, tmask, bb, t):
    h = jnp.tanh(
        lax.dot_general(x2, w1, (((1,), (0,)), ((), ())),
                        preferred_element_type=jnp.float32) + b1)
    s = jnp.sum(h * w2, axis=1) + b2
    s = s.reshape(bb, t)
    s = jnp.where(tmask, s, -1e9)
    m = jnp.max(s, axis=1, keepdims=True)
    e = jnp.exp(s - m)
    return e / jnp.sum(e, axis=1, keepdims=True)


def _score_kernel(x_ref, w1_ref, w2_ref, b1_ref, b2_ref, lens_ref,
                  score_ref, *, bb, t):
    lens_blk = lens_ref[...][:, 0]
    tmask = lax.broadcasted_iota(jnp.int32, (bb, t), 1) < lens_blk[:, None]
    width = x_ref.shape[-1]
    x2 = x_ref[...].reshape(bb * t, width)
    score_ref[...] = _attention(x2, w1_ref[...], w2_ref[...], b1_ref[...],
                                b2_ref[0, 0], tmask, bb, t)


def _fused_kernel(x_ref, w1_ref, w2_ref, b1_ref, b2_ref, lens_ref, tgt_ref,
                  out_ref, *, bb, t):
    lens_blk = lens_ref[...][:, 0]
    tmask = lax.broadcasted_iota(jnp.int32, (bb, t), 1) < lens_blk[:, None]
    width = x_ref.shape[-1]
    x = x_ref[...]
    p = _attention(x.reshape(bb * t, width), w1_ref[...], w2_ref[...],
                   b1_ref[...], b2_ref[0, 0], tmask, bb, t)
    tgt = tgt_ref[...][:, 0]
    onehot = (lax.broadcasted_iota(jnp.int32, (bb, width), 1)
              == tgt[:, None]).astype(jnp.float32)
    out_ref[...] = jnp.minimum(x + p[:, :, None] * onehot[:, None, :], 1.0)


def _sc_apply_body(x_hbm, score_hbm, tgt_hbm, out_hbm,
                   tgt_v, score_v, buf, *, t, width, rows_per_w, tch):
    nc = 2
    wid = lax.axis_index("s") * nc + lax.axis_index("c")
    base = wid * rows_per_w
    group16 = (wid * rows_per_w) // 16
    lane0 = base - group16 * 16
    pltpu.sync_copy(tgt_hbm.at[pl.ds(group16 * 16, 16)], tgt_v)
    row_idx = lax.broadcasted_iota(jnp.int32, (16,), 0)
    for r in range(rows_per_w):
        b = base + r
        pltpu.sync_copy(score_hbm.at[b], score_v)
        lane = jnp.full((16,), lane0 + r, dtype=jnp.int32)
        col_idx = plsc.load_gather(tgt_v, [lane])
        for c in range(t // tch):
            pltpu.sync_copy(x_hbm.at[b, pl.ds(c * tch, tch), :], buf)
            for u in range(tch // 16):
                s = score_v[pl.ds(c * tch + u * 16, 16)]
                ridx = row_idx + (u * 16)
                vals = plsc.load_gather(buf, [ridx, col_idx])
                plsc.store_scatter(buf, [ridx, col_idx],
                                   jnp.minimum(vals + s, 1.0))
            pltpu.sync_copy(buf, out_hbm.at[b, pl.ds(c * tch, tch), :])


def _make_sc_apply(b, t, width):
    rows_per_w = b // _NW
    mesh = plsc.VectorSubcoreMesh(core_axis_name="c", subcore_axis_name="s",
                                  num_cores=2, num_subcores=16)
    return pl.kernel(
        functools.partial(_sc_apply_body, t=t, width=width,
                          rows_per_w=rows_per_w, tch=_TCH),
        out_type=jax.ShapeDtypeStruct((b, t, width), jnp.float32),
        mesh=mesh,
        scratch_types=[
            pltpu.VMEM((16,), jnp.int32),
            pltpu.VMEM((t,), jnp.float32),
            pltpu.VMEM((_TCH, width), jnp.float32),
        ],
        compiler_params=pltpu.CompilerParams(needs_layout_passes=False),
    )


@jax.jit
def kernel(diagnosis_x, procedure_x, lens, target_diagnoses, target_procedures,
           Wd1, bd1, Wd2, bd2, Wp1, bp1, Wp2, bp2):
    b, t, dnum = diagnosis_x.shape
    pnum = procedure_x.shape[-1]
    adim = Wd1.shape[-1]
    bb = _BB
    grid = (b // bb,)

    lens2 = lens.astype(jnp.int32).reshape(b, 1)
    td2 = target_diagnoses.astype(jnp.int32).reshape(b, 1)
    tp1 = target_procedures.astype(jnp.int32)
    wd2r = Wd2.reshape(1, adim)
    wp2r = Wp2.reshape(1, adim)
    bd1r = bd1.reshape(1, adim)
    bp1r = bp1.reshape(1, adim)
    bd2r = bd2.reshape(1, 1)
    bp2r = bp2.reshape(1, 1)

    big = lambda w: pl.BlockSpec((bb, t, w), lambda i: (i, 0, 0))
    vfull = lambda shape: pl.BlockSpec(shape, lambda i: (0,) * len(shape))
    meta = pl.BlockSpec((bb, 1), lambda i: (i, 0))

    p_score = pl.pallas_call(
        functools.partial(_score_kernel, bb=bb, t=t),
        grid=grid,
        in_specs=[big(pnum), vfull((pnum, adim)), vfull((1, adim)),
                  vfull((1, adim)), vfull((1, 1)), meta],
        out_specs=pl.BlockSpec((bb, t), lambda i: (i, 0)),
        out_shape=jax.ShapeDtypeStruct((b, t), jnp.float32),
        compiler_params=pltpu.CompilerParams(
            dimension_semantics=("parallel",)),
    )(procedure_x, Wp1, wp2r, bp1r, bp2r, lens2)

    outp = _make_sc_apply(b, t, pnum)(procedure_x, p_score, tp1)

    outd = pl.pallas_call(
        functools.partial(_fused_kernel, bb=bb, t=t),
        grid=grid,
        in_specs=[big(dnum), vfull((dnum, adim)), vfull((1, adim)),
                  vfull((1, adim)), vfull((1, 1)), meta, meta],
        out_specs=big(dnum),
        out_shape=jax.ShapeDtypeStruct((b, t, dnum), jnp.float32),
        compiler_params=pltpu.CompilerParams(
            dimension_semantics=("parallel",)),
    )(diagnosis_x, Wd1, wd2r, bd1r, bd2r, lens2, td2)

    return (outd, outp)


# speedup vs baseline: 1.0004x; 1.0004x over previous
import functools

import jax
import jax.numpy as jnp
from jax import lax
from jax.experimental import pallas as pl
from jax.experimental.pallas import tpu as pltpu
from jax.experimental.pallas import tpu_sc as plsc

_BB = 8
_TCH = 16
_NW = 32


def _attention(x2, w1, w2, b1, b2, tmask, bb, t):
    h = jnp.tanh(
        lax.dot_general(x2, w1, (((1,), (0,)), ((), ())),
                        preferred_element_type=jnp.float32) + b1)
    s = jnp.sum(h * w2, axis=1) + b2
    s = s.reshape(bb, t)
    s = jnp.where(tmask, s, -1e9)
    m = jnp.max(s, axis=1, keepdims=True)
    e = jnp.exp(s - m)
    return e / jnp.sum(e, axis=1, keepdims=True)


def _score_kernel(x_ref, w1_ref, w2_ref, b1_ref, b2_ref, lens_ref,
                  score_ref, *, bb, t):
    lens_blk = lens_ref[...][:, 0]
    tmask = lax.broadcasted_iota(jnp.int32, (bb, t), 1) < lens_blk[:, None]
    width = x_ref.shape[-1]
    x2 = x_ref[...].reshape(bb * t, width)
    score_ref[...] = _attention(x2, w1_ref[...], w2_ref[...], b1_ref[...],
                                b2_ref[0, 0], tmask, bb, t)


def _fused_kernel(x_ref, w1_ref, w2_ref, b1_ref, b2_ref, lens_ref, tgt_ref,
                  out_ref, *, bb, t):
    lens_blk = lens_ref[...][:, 0]
    tmask = lax.broadcasted_iota(jnp.int32, (bb, t), 1) < lens_blk[:, None]
    width = x_ref.shape[-1]
    x = x_ref[...]
    p = _attention(x.reshape(bb * t, width), w1_ref[...], w2_ref[...],
                   b1_ref[...], b2_ref[0, 0], tmask, bb, t)
    tgt = tgt_ref[...][:, 0]
    onehot = (lax.broadcasted_iota(jnp.int32, (bb, width), 1)
              == tgt[:, None]).astype(jnp.float32)
    out_ref[...] = jnp.minimum(x + p[:, :, None] * onehot[:, None, :], 1.0)


def _sc_apply_body(x_hbm, score_hbm, tgt_hbm, out_hbm,
                   tgt_v, score_v, buf, *, t, width, rows_per_w, tch):
    nc = 2
    wid = lax.axis_index("s") * nc + lax.axis_index("c")
    base = wid * rows_per_w
    group16 = (wid * rows_per_w) // 16
    lane0 = base - group16 * 16
    pltpu.sync_copy(tgt_hbm.at[pl.ds(group16 * 16, 16)], tgt_v)
    row_idx = lax.broadcasted_iota(jnp.int32, (16,), 0)
    for r in range(rows_per_w):
        b = base + r
        pltpu.sync_copy(score_hbm.at[b], score_v)
        lane = jnp.full((16,), lane0 + r, dtype=jnp.int32)
        col_idx = plsc.load_gather(tgt_v, [lane])
        for c in range(t // tch):
            pltpu.sync_copy(x_hbm.at[b, pl.ds(c * tch, tch), :], buf)
            for u in range(tch // 16):
                s = score_v[pl.ds(c * tch + u * 16, 16)]
                ridx = row_idx + (u * 16)
                vals = plsc.load_gather(buf, [ridx, col_idx])
                plsc.store_scatter(buf, [ridx, col_idx],
                                   jnp.minimum(vals + s, 1.0))
            pltpu.sync_copy(buf, out_hbm.at[b, pl.ds(c * tch, tch), :])


def _make_sc_apply(b, t, width):
    rows_per_w = b // _NW
    mesh = plsc.VectorSubcoreMesh(core_axis_name="c", subcore_axis_name="s",
                                  num_cores=2, num_subcores=16)
    return pl.kernel(
        functools.partial(_sc_apply_body, t=t, width=width,
                          rows_per_w=rows_per_w, tch=_TCH),
        out_type=jax.ShapeDtypeStruct((b, t, width), jnp.float32),
        mesh=mesh,
        scratch_types=[
            pltpu.VMEM((16,), jnp.int32),
            pltpu.VMEM((t,), jnp.float32),
            pltpu.VMEM((_TCH, width), jnp.float32),
        ],
        compiler_params=pltpu.CompilerParams(needs_layout_passes=False),
    )


@jax.jit
def kernel(diagnosis_x, procedure_x, lens, target_diagnoses, target_procedures,
           Wd1, bd1, Wd2, bd2, Wp1, bp1, Wp2, bp2):
    b, t, dnum = diagnosis_x.shape
    pnum = procedure_x.shape[-1]
    adim = Wd1.shape[-1]
    bb = _BB
    grid = (b // bb,)

    lens2 = lens.astype(jnp.int32).reshape(b, 1)
    td2 = target_diagnoses.astype(jnp.int32).reshape(b, 1)
    tp1 = target_procedures.astype(jnp.int32)
    wd2r = Wd2.reshape(1, adim)
    wp2r = Wp2.reshape(1, adim)
    bd1r = bd1.reshape(1, adim)
    bp1r = bp1.reshape(1, adim)
    bd2r = bd2.reshape(1, 1)
    bp2r = bp2.reshape(1, 1)

    big = lambda w: pl.BlockSpec((bb, t, w), lambda i: (i, 0, 0))
    vfull = lambda shape: pl.BlockSpec(shape, lambda i: (0,) * len(shape))
    meta = pl.BlockSpec((bb, 1), lambda i: (i, 0))

    p_score = pl.pallas_call(
        functools.partial(_score_kernel, bb=bb, t=t),
        grid=grid,
        in_specs=[big(pnum), vfull((pnum, adim)), vfull((1, adim)),
                  vfull((1, adim)), vfull((1, 1)), meta],
        out_specs=pl.BlockSpec((bb, t), lambda i: (i, 0)),
        out_shape=jax.ShapeDtypeStruct((b, t), jnp.float32),
        compiler_params=pltpu.CompilerParams(
            dimension_semantics=("parallel",)),
    )(procedure_x, Wp1, wp2r, bp1r, bp2r, lens2)

    outd = pl.pallas_call(
        functools.partial(_fused_kernel, bb=bb, t=t),
        grid=grid,
        in_specs=[big(dnum), vfull((dnum, adim)), vfull((1, adim)),
                  vfull((1, adim)), vfull((1, 1)), meta, meta],
        out_specs=big(dnum),
        out_shape=jax.ShapeDtypeStruct((b, t, dnum), jnp.float32),
        compiler_params=pltpu.CompilerParams(
            dimension_semantics=("parallel",)),
    )(diagnosis_x, Wd1, wd2r, bd1r, bd2r, lens2, td2)

    outp = _make_sc_apply(b, t, pnum)(procedure_x, p_score, tp1)

    return (outd, outp)


# speedup vs baseline: 1.0269x; 1.0265x over previous
import functools

import jax
import jax.numpy as jnp
from jax import lax
from jax.experimental import pallas as pl
from jax.experimental.pallas import tpu as pltpu
from jax.experimental.pallas import tpu_sc as plsc

_BBL = 128


def _passA_diag_kernel(x_ref, w1_ref, w2_ref, b1_ref, b2_ref, lens_ref,
                       out_ref, score_ref, h_acc, *, t, nd):
    j = pl.program_id(1)
    x = x_ref[...]
    out_ref[...] = jnp.minimum(x, 1.0)
    partial = lax.dot_general(w1_ref[...], x, (((0,), (1,)), ((), ())),
                              preferred_element_type=jnp.float32)

    @pl.when(j == 0)
    def _():
        h_acc[...] = partial

    @pl.when(j > 0)
    def _():
        h_acc[...] += partial

    @pl.when(j == nd - 1)
    def _():
        h = jnp.tanh(h_acc[...] + b1_ref[...][:, :, None])
        s = jnp.sum(h * w2_ref[...][:, :, None], axis=0) + b2_ref[0, 0]
        lens_blk = lens_ref[...][0, 0, :]
        tmask = (lax.broadcasted_iota(jnp.int32, (t, s.shape[-1]), 0)
                 < lens_blk[None, :])
        s = jnp.where(tmask, s, -1e9)
        m = jnp.max(s, axis=0, keepdims=True)
        e = jnp.exp(s - m)
        p = e / jnp.sum(e, axis=0, keepdims=True)
        score_ref[...] = jnp.swapaxes(p, 0, 1)


def _passA_proc_kernel(x_ref, w1_ref, w2_ref, b1_ref, b2_ref, lens_ref,
                       out_ref, score_ref, h_acc, *, t, nd):
    j = pl.program_id(1)
    x = x_ref[...]
    out_ref[...] = jnp.minimum(x, 1.0)
    pc = x.shape[0]
    w1c = w1_ref[pl.ds(j * pc, pc), :]
    partial = lax.dot_general(w1c, x, (((0,), (0,)), ((), ())),
                              preferred_element_type=jnp.float32)

    @pl.when(j == 0)
    def _():
        h_acc[...] = partial

    @pl.when(j > 0)
    def _():
        h_acc[...] += partial

    @pl.when(j == nd - 1)
    def _():
        h = jnp.tanh(h_acc[...] + b1_ref[...][:, :, None])
        s = jnp.sum(h * w2_ref[...][:, :, None], axis=0) + b2_ref[0, 0]
        lens_blk = lens_ref[...][0, 0, :]
        tmask = (lax.broadcasted_iota(jnp.int32, (t, s.shape[-1]), 0)
                 < lens_blk[None, :])
        s = jnp.where(tmask, s, -1e9)
        m = jnp.max(s, axis=0, keepdims=True)
        e = jnp.exp(s - m)
        p = e / jnp.sum(e, axis=0, keepdims=True)
        score_ref[...] = jnp.swapaxes(p, 0, 1)


def _passA(x_v, w1, w2c, b1c, b2c, lens3, *, kernel_fn, chunk, chunk_axis,
           t, b, bbl):
    nd = x_v.shape[chunk_axis] // chunk
    nb = b // bbl
    adim = w1.shape[1]
    if chunk_axis == 1:
        big = pl.BlockSpec((t, chunk, bbl), lambda i, j: (0, j, i))
        w1spec = pl.BlockSpec((chunk, adim), lambda i, j: (j, 0))
    else:
        big = pl.BlockSpec((chunk, t, bbl), lambda i, j: (j, 0, i))
        w1spec = pl.BlockSpec(w1.shape, lambda i, j: (0, 0))
    out_v, score = pl.pallas_call(
        functools.partial(kernel_fn, t=t, nd=nd),
        grid=(nb, nd),
        in_specs=[
            big,
            w1spec,
            pl.BlockSpec((adim, 1), lambda i, j: (0, 0)),
            pl.BlockSpec((adim, 1), lambda i, j: (0, 0)),
            pl.BlockSpec((1, 1), lambda i, j: (0, 0)),
            pl.BlockSpec((1, 1, bbl), lambda i, j: (i, 0, 0)),
        ],
        out_specs=[
            big,
            pl.BlockSpec((bbl, t), lambda i, j: (i, 0)),
        ],
        out_shape=[
            jax.ShapeDtypeStruct(x_v.shape, jnp.float32),
            jax.ShapeDtypeStruct((b, t), jnp.float32),
        ],
        scratch_shapes=[pltpu.VMEM((adim, t, bbl), jnp.float32)],
        compiler_params=pltpu.CompilerParams(
            dimension_semantics=("arbitrary", "arbitrary")),
    )(x_v, w1, w2c, b1c, b2c, lens3)
    return out_v, score


def _sc_fix_body(outd_ref, outp_ref, sd_hbm, sp_hbm, td_hbm, tp_hbm,
                 tgtd_v, tgtp_v, score_v, idx_v, val_v, sem,
                 *, t, b, dnum, pnum, rows_per_w):
    nc = 2
    wid = lax.axis_index("s") * nc + lax.axis_index("c")
    base = wid * rows_per_w
    group16 = base // 16
    lane0 = base - group16 * 16
    pltpu.sync_copy(td_hbm.at[pl.ds(group16 * 16, 16)], tgtd_v)
    pltpu.sync_copy(tp_hbm.at[pl.ds(group16 * 16, 16)], tgtp_v)
    iota16 = lax.broadcasted_iota(jnp.int32, (16,), 0)

    def branch(out_ref, s_hbm, tgt_ref, width):
        if width == dnum:
            t_stride, c_stride = width * b, b
        else:
            t_stride, c_stride = b, t * b
        for r in range(rows_per_w):
            row = base + r
            pltpu.sync_copy(s_hbm.at[row], score_v)
            lane = jnp.full((16,), lane0 + r, dtype=jnp.int32)
            tgt = plsc.load_gather(tgt_ref, [lane])
            for u in range(t // 16):
                tvec = iota16 + (u * 16)
                idx = tvec * t_stride + tgt * c_stride + row
                idx_v[pl.ds(u * 16, 16)] = idx
            pltpu.make_async_copy(out_ref.at[idx_v], val_v, sem).start()
            pltpu.make_async_copy(out_ref.at[idx_v], val_v, sem).wait()
            for u in range(t // 16):
                s = score_v[pl.ds(u * 16, 16)]
                v = val_v[pl.ds(u * 16, 16)]
                val_v[pl.ds(u * 16, 16)] = jnp.minimum(v + s, 1.0)
            pltpu.make_async_copy(val_v, out_ref.at[idx_v], sem).start()
            pltpu.make_async_copy(val_v, out_ref.at[idx_v], sem).wait()

    branch(outd_ref, sd_hbm, tgtd_v, dnum)
    branch(outp_ref, sp_hbm, tgtp_v, pnum)


def _make_sc_fix(b, t, dnum, pnum):
    rows_per_w = b // 32
    mesh = plsc.VectorSubcoreMesh(core_axis_name="c", subcore_axis_name="s",
                                  num_cores=2, num_subcores=16)
    return pl.kernel(
        functools.partial(_sc_fix_body, t=t, b=b, dnum=dnum, pnum=pnum,
                          rows_per_w=rows_per_w),
        out_type=(),
        mesh=mesh,
        scratch_types=[
            pltpu.VMEM((16,), jnp.int32),
            pltpu.VMEM((16,), jnp.int32),
            pltpu.VMEM((t,), jnp.float32),
            pltpu.VMEM((t,), jnp.int32),
            pltpu.VMEM((t,), jnp.float32),
            pltpu.SemaphoreType.DMA,
        ],
        compiler_params=pltpu.CompilerParams(needs_layout_passes=False),
    )


@jax.jit
def kernel(diagnosis_x, procedure_x, lens, target_diagnoses, target_procedures,
           Wd1, bd1, Wd2, bd2, Wp1, bp1, Wp2, bp2):
    b, t, dnum = diagnosis_x.shape
    pnum = procedure_x.shape[-1]
    adim = Wd1.shape[-1]

    xd_v = jnp.transpose(diagnosis_x, (1, 2, 0))
    xp_v = jnp.transpose(procedure_x, (2, 1, 0))

    lens3 = lens.astype(jnp.int32).reshape(b // _BBL, 1, _BBL)
    td1 = target_diagnoses.astype(jnp.int32)
    tp1 = target_procedures.astype(jnp.int32)

    outd_v, d_score = _passA(
        xd_v, Wd1, Wd2.reshape(adim, 1), bd1.reshape(adim, 1),
        bd2.reshape(1, 1), lens3,
        kernel_fn=_passA_diag_kernel, chunk=200, chunk_axis=1,
        t=t, b=b, bbl=_BBL)
    outp_v, p_score = _passA(
        xp_v, Wp1, Wp2.reshape(adim, 1), bp1.reshape(adim, 1),
        bp2.reshape(1, 1), lens3,
        kernel_fn=_passA_proc_kernel, chunk=250, chunk_axis=0,
        t=t, b=b, bbl=_BBL)

    refd = jax.new_ref(outd_v.reshape(-1))
    refp = jax.new_ref(outp_v.reshape(-1))
    _make_sc_fix(b, t, dnum, pnum)(refd, refp, d_score, p_score, td1, tp1)
    outd_v2 = refd[...].reshape(t, dnum, b)
    outp_v2 = refp[...].reshape(pnum, t, b)

    outd = jnp.transpose(outd_v2, (2, 0, 1))
    outp = jnp.transpose(outp_v2, (2, 1, 0))
    return (outd, outp)


# speedup vs baseline: 2.8444x; 2.7698x over previous
import functools

import jax
import jax.numpy as jnp
from jax import lax
from jax.experimental import pallas as pl
from jax.experimental.pallas import tpu as pltpu
from jax.experimental.pallas import tpu_sc as plsc

_BBL = 128


def _passA_diag_kernel(x_ref, w1_ref, w2_ref, b1_ref, b2_ref, lens_ref,
                       out_ref, score_ref, h_acc, *, t, nd):
    j = pl.program_id(1)
    x = x_ref[...]
    out_ref[...] = jnp.minimum(x, 1.0)
    partial = lax.dot_general(w1_ref[...], x, (((0,), (1,)), ((), ())),
                              preferred_element_type=jnp.float32)

    @pl.when(j == 0)
    def _():
        h_acc[...] = partial

    @pl.when(j > 0)
    def _():
        h_acc[...] += partial

    @pl.when(j == nd - 1)
    def _():
        h = jnp.tanh(h_acc[...] + b1_ref[...][:, :, None])
        s = jnp.sum(h * w2_ref[...][:, :, None], axis=0) + b2_ref[0, 0]
        lens_blk = lens_ref[...][0, 0, :]
        tmask = (lax.broadcasted_iota(jnp.int32, (t, s.shape[-1]), 0)
                 < lens_blk[None, :])
        s = jnp.where(tmask, s, -1e9)
        m = jnp.max(s, axis=0, keepdims=True)
        e = jnp.exp(s - m)
        p = e / jnp.sum(e, axis=0, keepdims=True)
        score_ref[...] = jnp.swapaxes(p, 0, 1)


def _passA_proc_kernel(x_ref, w1_ref, w2_ref, b1_ref, b2_ref, lens_ref,
                       out_ref, score_ref, h_acc, *, t, nd):
    j = pl.program_id(1)
    x = x_ref[...]
    out_ref[...] = jnp.minimum(x, 1.0)
    pc = x.shape[0]
    w1c = w1_ref[pl.ds(j * pc, pc), :]
    partial = lax.dot_general(w1c, x, (((0,), (0,)), ((), ())),
                              preferred_element_type=jnp.float32)

    @pl.when(j == 0)
    def _():
        h_acc[...] = partial

    @pl.when(j > 0)
    def _():
        h_acc[...] += partial

    @pl.when(j == nd - 1)
    def _():
        h = jnp.tanh(h_acc[...] + b1_ref[...][:, :, None])
        s = jnp.sum(h * w2_ref[...][:, :, None], axis=0) + b2_ref[0, 0]
        lens_blk = lens_ref[...][0, 0, :]
        tmask = (lax.broadcasted_iota(jnp.int32, (t, s.shape[-1]), 0)
                 < lens_blk[None, :])
        s = jnp.where(tmask, s, -1e9)
        m = jnp.max(s, axis=0, keepdims=True)
        e = jnp.exp(s - m)
        p = e / jnp.sum(e, axis=0, keepdims=True)
        score_ref[...] = jnp.swapaxes(p, 0, 1)


def _passA(x_v, w1, w2c, b1c, b2c, lens3, *, kernel_fn, chunk, chunk_axis,
           t, b, bbl):
    nd = x_v.shape[chunk_axis] // chunk
    nb = b // bbl
    adim = w1.shape[1]
    if chunk_axis == 1:
        big = pl.BlockSpec((t, chunk, bbl), lambda i, j: (0, j, i))
        w1spec = pl.BlockSpec((chunk, adim), lambda i, j: (j, 0))
    else:
        big = pl.BlockSpec((chunk, t, bbl), lambda i, j: (j, 0, i))
        w1spec = pl.BlockSpec(w1.shape, lambda i, j: (0, 0))
    out_v, score = pl.pallas_call(
        functools.partial(kernel_fn, t=t, nd=nd),
        grid=(nb, nd),
        in_specs=[
            big,
            w1spec,
            pl.BlockSpec((adim, 1), lambda i, j: (0, 0)),
            pl.BlockSpec((adim, 1), lambda i, j: (0, 0)),
            pl.BlockSpec((1, 1), lambda i, j: (0, 0)),
            pl.BlockSpec((1, 1, bbl), lambda i, j: (i, 0, 0)),
        ],
        out_specs=[
            big,
            pl.BlockSpec((bbl, t), lambda i, j: (i, 0)),
        ],
        out_shape=[
            jax.ShapeDtypeStruct(x_v.shape, jnp.float32),
            jax.ShapeDtypeStruct((b, t), jnp.float32),
        ],
        scratch_shapes=[pltpu.VMEM((adim, t, bbl), jnp.float32)],
        compiler_params=pltpu.CompilerParams(
            dimension_semantics=("arbitrary", "arbitrary")),
    )(x_v, w1, w2c, b1c, b2c, lens3)
    return out_v, score


def _sc_fix_body(outd_ref, outp_ref, sd_hbm, sp_hbm, td_hbm, tp_hbm,
                 tgtd_v, tgtp_v, score_v, idx_v, val_v, sem,
                 *, t, b, dnum, pnum, rows_per_w):
    nc = 2
    wid = lax.axis_index("s") * nc + lax.axis_index("c")
    base = wid * rows_per_w
    group16 = base // 16
    lane0 = base - group16 * 16
    pltpu.sync_copy(td_hbm.at[pl.ds(group16 * 16, 16)], tgtd_v)
    pltpu.sync_copy(tp_hbm.at[pl.ds(group16 * 16, 16)], tgtp_v)
    iota16 = lax.broadcasted_iota(jnp.int32, (16,), 0)

    def branch(out_ref, s_hbm, tgt_ref, width, is_diag):
        bt = b // 128
        for r in range(rows_per_w):
            row = base + r
            b_hi = (row // 128) * 1024
            b_lo = row - (row // 128) * 128
            pltpu.sync_copy(s_hbm.at[row], score_v)
            lane = jnp.full((16,), lane0 + r, dtype=jnp.int32)
            tgt = plsc.load_gather(tgt_ref, [lane])
            for u in range(t // 16):
                tvec = iota16 + (u * 16)
                if is_diag:
                    idx = (tvec * ((width // 8) * bt * 1024)
                           + (tgt // 8) * (bt * 1024)
                           + (tgt % 8) * 128 + (b_hi + b_lo))
                else:
                    idx = (tgt * (8 * bt * 1024)
                           + (tvec // 8) * (bt * 1024)
                           + (tvec % 8) * 128 + (b_hi + b_lo))
                idx_v[pl.ds(u * 16, 16)] = idx
            pltpu.make_async_copy(out_ref.at[idx_v], val_v, sem).start()
            pltpu.make_async_copy(out_ref.at[idx_v], val_v, sem).wait()
            for u in range(t // 16):
                sc = score_v[pl.ds(u * 16, 16)]
                v = val_v[pl.ds(u * 16, 16)]
                val_v[pl.ds(u * 16, 16)] = jnp.minimum(v + sc, 1.0)
            pltpu.make_async_copy(val_v, out_ref.at[idx_v], sem).start()
            pltpu.make_async_copy(val_v, out_ref.at[idx_v], sem).wait()

    branch(outd_ref, sd_hbm, tgtd_v, dnum, True)
    branch(outp_ref, sp_hbm, tgtp_v, pnum, False)


def _make_sc_fix(b, t, dnum, pnum):
    rows_per_w = b // 32
    mesh = plsc.VectorSubcoreMesh(core_axis_name="c", subcore_axis_name="s",
                                  num_cores=2, num_subcores=16)
    return pl.kernel(
        functools.partial(_sc_fix_body, t=t, b=b, dnum=dnum, pnum=pnum,
                          rows_per_w=rows_per_w),
        out_type=(),
        mesh=mesh,
        scratch_types=[
            pltpu.VMEM((16,), jnp.int32),
            pltpu.VMEM((16,), jnp.int32),
            pltpu.VMEM((t,), jnp.float32),
            pltpu.VMEM((t,), jnp.int32),
            pltpu.VMEM((t,), jnp.float32),
            pltpu.SemaphoreType.DMA,
        ],
        compiler_params=pltpu.CompilerParams(needs_layout_passes=False),
    )


@jax.jit
def kernel(diagnosis_x, procedure_x, lens, target_diagnoses, target_procedures,
           Wd1, bd1, Wd2, bd2, Wp1, bp1, Wp2, bp2):
    b, t, dnum = diagnosis_x.shape
    pnum = procedure_x.shape[-1]
    adim = Wd1.shape[-1]

    xd_v = jnp.transpose(diagnosis_x, (1, 2, 0))
    xp_v = jnp.transpose(procedure_x, (2, 1, 0))

    lens3 = lens.astype(jnp.int32).reshape(b // _BBL, 1, _BBL)
    td1 = target_diagnoses.astype(jnp.int32)
    tp1 = target_procedures.astype(jnp.int32)

    outd_v, d_score = _passA(
        xd_v, Wd1, Wd2.reshape(adim, 1), bd1.reshape(adim, 1),
        bd2.reshape(1, 1), lens3,
        kernel_fn=_passA_diag_kernel, chunk=200, chunk_axis=1,
        t=t, b=b, bbl=_BBL)
    outp_v, p_score = _passA(
        xp_v, Wp1, Wp2.reshape(adim, 1), bp1.reshape(adim, 1),
        bp2.reshape(1, 1), lens3,
        kernel_fn=_passA_proc_kernel, chunk=250, chunk_axis=0,
        t=t, b=b, bbl=_BBL)

    outd_f = jnp.transpose(
        outd_v.reshape(t, dnum // 8, 8, b // 128, 128),
        (0, 1, 3, 2, 4)).reshape(-1)
    outp_f = jnp.transpose(
        outp_v.reshape(pnum, 8, 8, b // 128, 128),
        (0, 1, 3, 2, 4)).reshape(-1)
    refd = jax.new_ref(outd_f)
    refp = jax.new_ref(outp_f)
    _make_sc_fix(b, t, dnum, pnum)(refd, refp, d_score, p_score, td1, tp1)
    outd_v2 = jnp.transpose(
        refd[...].reshape(t, dnum // 8, b // 128, 8, 128),
        (0, 1, 3, 2, 4)).reshape(t, dnum, b)
    outp_v2 = jnp.transpose(
        refp[...].reshape(pnum, 8, b // 128, 8, 128),
        (0, 1, 3, 2, 4)).reshape(pnum, t, b)

    outd = jnp.transpose(outd_v2, (2, 0, 1))
    outp = jnp.transpose(outp_v2, (2, 1, 0))
    return (outd, outp)


# speedup vs baseline: 2.8556x; 1.0040x over previous
import functools

import jax
import jax.numpy as jnp
from jax import lax
from jax.experimental import pallas as pl
from jax.experimental.pallas import tpu as pltpu
from jax.experimental.pallas import tpu_sc as plsc

_BBL = 128


def _passA_diag_kernel(x_ref, w1_ref, w2_ref, b1_ref, b2_ref, lens_ref,
                       out_ref, score_ref, h_acc, *, t, nd):
    j = pl.program_id(1)
    x = x_ref[...]
    out_ref[...] = jnp.minimum(x, 1.0)
    partial = lax.dot_general(w1_ref[...], x, (((0,), (1,)), ((), ())),
                              preferred_element_type=jnp.float32)

    @pl.when(j == 0)
    def _():
        h_acc[...] = partial

    @pl.when(j > 0)
    def _():
        h_acc[...] += partial

    @pl.when(j == nd - 1)
    def _():
        h = jnp.tanh(h_acc[...] + b1_ref[...][:, :, None])
        s = jnp.sum(h * w2_ref[...][:, :, None], axis=0) + b2_ref[0, 0]
        lens_blk = lens_ref[...][0, 0, :]
        tmask = (lax.broadcasted_iota(jnp.int32, (t, s.shape[-1]), 0)
                 < lens_blk[None, :])
        s = jnp.where(tmask, s, -1e9)
        m = jnp.max(s, axis=0, keepdims=True)
        e = jnp.exp(s - m)
        p = e / jnp.sum(e, axis=0, keepdims=True)
        score_ref[...] = jnp.swapaxes(p, 0, 1)


def _passA_proc_kernel(x_ref, w1_ref, w2_ref, b1_ref, b2_ref, lens_ref,
                       out_ref, score_ref, h_acc, *, t, nd):
    j = pl.program_id(1)
    x = x_ref[...]
    out_ref[...] = jnp.minimum(x, 1.0)
    pc = x.shape[0]
    w1c = w1_ref[pl.ds(j * pc, pc), :]
    partial = lax.dot_general(w1c, x, (((0,), (0,)), ((), ())),
                              preferred_element_type=jnp.float32)

    @pl.when(j == 0)
    def _():
        h_acc[...] = partial

    @pl.when(j > 0)
    def _():
        h_acc[...] += partial

    @pl.when(j == nd - 1)
    def _():
        h = jnp.tanh(h_acc[...] + b1_ref[...][:, :, None])
        s = jnp.sum(h * w2_ref[...][:, :, None], axis=0) + b2_ref[0, 0]
        lens_blk = lens_ref[...][0, 0, :]
        tmask = (lax.broadcasted_iota(jnp.int32, (t, s.shape[-1]), 0)
                 < lens_blk[None, :])
        s = jnp.where(tmask, s, -1e9)
        m = jnp.max(s, axis=0, keepdims=True)
        e = jnp.exp(s - m)
        p = e / jnp.sum(e, axis=0, keepdims=True)
        score_ref[...] = jnp.swapaxes(p, 0, 1)


def _passA(x_v, w1, w2c, b1c, b2c, lens3, *, kernel_fn, chunk, chunk_axis,
           t, b, bbl):
    nd = x_v.shape[chunk_axis] // chunk
    nb = b // bbl
    adim = w1.shape[1]
    if chunk_axis == 1:
        big = pl.BlockSpec((t, chunk, bbl), lambda i, j: (0, j, i))
        w1spec = pl.BlockSpec((chunk, adim), lambda i, j: (j, 0))
    else:
        big = pl.BlockSpec((chunk, t, bbl), lambda i, j: (j, 0, i))
        w1spec = pl.BlockSpec(w1.shape, lambda i, j: (0, 0))
    out_v, score = pl.pallas_call(
        functools.partial(kernel_fn, t=t, nd=nd),
        grid=(nb, nd),
        in_specs=[
            big,
            w1spec,
            pl.BlockSpec((adim, 1), lambda i, j: (0, 0)),
            pl.BlockSpec((adim, 1), lambda i, j: (0, 0)),
            pl.BlockSpec((1, 1), lambda i, j: (0, 0)),
            pl.BlockSpec((1, 1, bbl), lambda i, j: (i, 0, 0)),
        ],
        out_specs=[
            big,
            pl.BlockSpec((bbl, t), lambda i, j: (i, 0)),
        ],
        out_shape=[
            jax.ShapeDtypeStruct(x_v.shape, jnp.float32),
            jax.ShapeDtypeStruct((b, t), jnp.float32),
        ],
        scratch_shapes=[pltpu.VMEM((adim, t, bbl), jnp.float32)],
        compiler_params=pltpu.CompilerParams(
            dimension_semantics=("arbitrary", "arbitrary")),
    )(x_v, w1, w2c, b1c, b2c, lens3)
    return out_v, score


def _sc_fix_body(outd_ref, outp_ref, sd_hbm, sp_hbm, td_hbm, tp_hbm,
                 tgtd_v, tgtp_v, score_v, idx_v, val_v, sem,
                 *, t, b, dnum, pnum, rows_per_w):
    nc = 2
    wid = lax.axis_index("s") * nc + lax.axis_index("c")
    base = wid * rows_per_w
    group16 = base // 16
    lane0 = base - group16 * 16
    pltpu.sync_copy(td_hbm.at[pl.ds(group16 * 16, 16)], tgtd_v)
    pltpu.sync_copy(tp_hbm.at[pl.ds(group16 * 16, 16)], tgtp_v)
    iota16 = lax.broadcasted_iota(jnp.int32, (16,), 0)

    def branch(out_ref, s_hbm, tgt_ref, width, is_diag):
        bt = b // 128
        for r in range(rows_per_w):
            row = base + r
            b_hi = (row // 128) * 1024
            b_lo = row - (row // 128) * 128
            pltpu.sync_copy(s_hbm.at[row], score_v)
            lane = jnp.full((16,), lane0 + r, dtype=jnp.int32)
            tgt = plsc.load_gather(tgt_ref, [lane])
            for u in range(t // 16):
                tvec = iota16 + (u * 16)
                if is_diag:
                    idx = (tvec * ((width // 8) * bt * 1024)
                           + (tgt // 8) * (bt * 1024)
                           + (tgt % 8) * 128 + (b_hi + b_lo))
                else:
                    idx = (tgt * (8 * bt * 1024)
                           + (tvec // 8) * (bt * 1024)
                           + (tvec % 8) * 128 + (b_hi + b_lo))
                idx_v[pl.ds(u * 16, 16)] = idx
            pltpu.make_async_copy(out_ref.at[idx_v], val_v, sem).start()
            pltpu.make_async_copy(out_ref.at[idx_v], val_v, sem).wait()
            for u in range(t // 16):
                sc = score_v[pl.ds(u * 16, 16)]
                v = val_v[pl.ds(u * 16, 16)]
                val_v[pl.ds(u * 16, 16)] = jnp.minimum(v + sc, 1.0)
            pltpu.make_async_copy(val_v, out_ref.at[idx_v], sem).start()
            pltpu.make_async_copy(val_v, out_ref.at[idx_v], sem).wait()

    branch(outd_ref, sd_hbm, tgtd_v, dnum, True)
    branch(outp_ref, sp_hbm, tgtp_v, pnum, False)


def _make_sc_fix(b, t, dnum, pnum):
    rows_per_w = b // 32
    mesh = plsc.VectorSubcoreMesh(core_axis_name="c", subcore_axis_name="s",
                                  num_cores=2, num_subcores=16)
    return pl.kernel(
        functools.partial(_sc_fix_body, t=t, b=b, dnum=dnum, pnum=pnum,
                          rows_per_w=rows_per_w),
        out_type=(),
        mesh=mesh,
        scratch_types=[
            pltpu.VMEM((16,), jnp.int32),
            pltpu.VMEM((16,), jnp.int32),
            pltpu.VMEM((t,), jnp.float32),
            pltpu.VMEM((t,), jnp.int32),
            pltpu.VMEM((t,), jnp.float32),
            pltpu.SemaphoreType.DMA,
        ],
        compiler_params=pltpu.CompilerParams(needs_layout_passes=False),
    )


@jax.jit
def kernel(diagnosis_x, procedure_x, lens, target_diagnoses, target_procedures,
           Wd1, bd1, Wd2, bd2, Wp1, bp1, Wp2, bp2):
    b, t, dnum = diagnosis_x.shape
    pnum = procedure_x.shape[-1]
    adim = Wd1.shape[-1]

    xd_v = jnp.transpose(diagnosis_x, (1, 2, 0))
    xp_v = jnp.transpose(procedure_x, (2, 1, 0))

    lens3 = lens.astype(jnp.int32).reshape(b // _BBL, 1, _BBL)
    td1 = target_diagnoses.astype(jnp.int32)
    tp1 = target_procedures.astype(jnp.int32)

    outd_v, d_score = _passA(
        xd_v, Wd1, Wd2.reshape(adim, 1), bd1.reshape(adim, 1),
        bd2.reshape(1, 1), lens3,
        kernel_fn=_passA_diag_kernel, chunk=400, chunk_axis=1,
        t=t, b=b, bbl=_BBL)
    outp_v, p_score = _passA(
        xp_v, Wp1, Wp2.reshape(adim, 1), bp1.reshape(adim, 1),
        bp2.reshape(1, 1), lens3,
        kernel_fn=_passA_proc_kernel, chunk=375, chunk_axis=0,
        t=t, b=b, bbl=_BBL)

    outd_f = jnp.transpose(
        outd_v.reshape(t, dnum // 8, 8, b // 128, 128),
        (0, 1, 3, 2, 4)).reshape(-1)
    outp_f = jnp.transpose(
        outp_v.reshape(pnum, 8, 8, b // 128, 128),
        (0, 1, 3, 2, 4)).reshape(-1)
    refd = jax.new_ref(outd_f)
    refp = jax.new_ref(outp_f)
    _make_sc_fix(b, t, dnum, pnum)(refd, refp, d_score, p_score, td1, tp1)
    outd_v2 = jnp.transpose(
        refd[...].reshape(t, dnum // 8, b // 128, 8, 128),
        (0, 1, 3, 2, 4)).reshape(t, dnum, b)
    outp_v2 = jnp.transpose(
        refp[...].reshape(pnum, 8, b // 128, 8, 128),
        (0, 1, 3, 2, 4)).reshape(pnum, t, b)

    outd = jnp.transpose(outd_v2, (2, 0, 1))
    outp = jnp.transpose(outp_v2, (2, 1, 0))
    return (outd, outp)
